# baseline (speedup 1.0000x reference)
import functools

import jax
import jax.numpy as jnp
from jax import lax
from jax.experimental import pallas as pl
from jax.experimental.pallas import tpu as pltpu
from jax.experimental.pallas import tpu_sc as plsc

_N = 4096
_K = 32
_H = 64
_WROWS = 9
_WCAND = _WROWS * _H
_C = 128
_NKNOTS = 32


def _rne(a):
    b = lax.bitcast_convert_type(a, jnp.int32)
    r = (b + jnp.int32(0x7FFF) + ((b >> 16) & 1)) & jnp.int32(-65536)
    return lax.bitcast_convert_type(r, jnp.float32)


def _nbr_body(lin_smem, knots_smem, smx_smem, smy_smem,
              lin_col, phiw, phii, phij,
              jout, prod_out, psis_out, phis_out):
    r0 = pl.program_id(0)
    start = jnp.clip(r0 - 4, 0, _H - _WROWS)
    lin = lin_col[...]
    riota = lax.broadcasted_iota(jnp.int32, (_H, _WCAND), 0)
    liota = lax.broadcasted_iota(jnp.int32, (_H, _WCAND), 1)
    lin_b = jnp.broadcast_to(lin, (_H, _WCAND))
    yc = jnp.sum(jnp.where(liota % _H == riota, lin_b, 0.0), axis=0,
                 keepdims=True)
    xc = jnp.sum(jnp.where(liota // _H + start == riota, lin_b, 0.0), axis=0,
                 keepdims=True)
    xi = lin_smem[r0]
    dx = xi - xc
    dy = lin - yc
    dist = jnp.sqrt(dx * dx + dy * dy)
    sel = []
    for _ in range(_K):
        m = jnp.min(dist, axis=1, keepdims=True)
        cand = jnp.where(dist == m, liota, jnp.int32(1 << 30))
        amin = jnp.min(cand, axis=1, keepdims=True)
        sel.append(amin)
        dist = jnp.where(cand == amin, jnp.inf, dist)
    lsel = jnp.concatenate(sel, axis=1)
    jout[0] = start * _H + lsel
    rsel = lsel // _H
    csel = lsel % _H
    cjx = jnp.zeros((_H, _K), jnp.float32)
    cjy = jnp.zeros((_H, _K), jnp.float32)
    for t in range(_WROWS):
        cjx = jnp.where(rsel == t, lin_smem[start + t], cjx)
    for c in range(_H):
        cjy = jnp.where(csel == c, lin_smem[c], cjy)
    relx = xi - cjx
    rely = lin - cjy
    psi_x = jnp.zeros((_H, _K), jnp.float32)
    psi_y = jnp.zeros((_H, _K), jnp.float32)
    for t in range(_NKNOTS):
        kv = knots_smem[t]
        psi_x = psi_x + jnp.maximum(1.0 - jnp.abs(relx - kv), 0.0) * smx_smem[t]
        psi_y = psi_y + jnp.maximum(1.0 - jnp.abs(rely - kv), 0.0) * smy_smem[t]
    psi = psi_x * psi_y
    rpw = _rne(phiw[...])
    rpi = _rne(phii[...])
    rpj = _rne(phij[...])
    i_e = _rne(xi) * rpi[0:1, :] + _rne(lin) * rpi[1:2, :]
    j_e = (_rne(cjx)[:, :, None] * rpj[0:1, :].reshape(1, 1, _C)
           + _rne(cjy)[:, :, None] * rpj[1:2, :].reshape(1, 1, _C))
    t = i_e[:, None, :] * j_e
    phiv = jnp.sum(_rne(t) * rpw.reshape(1, 1, _C), axis=2)
    prod_out[0] = psi * phiv
    psis_out[0] = jnp.sum(jnp.abs(psi)).reshape(1, 1)
    phis_out[0] = jnp.sum(jnp.abs(phiv)).reshape(1, 1)


def _neighbors_and_weights(lin, knots, S_m_x, S_m_y, phi_w, phi_i, phi_j):
    out_shapes = [
        jax.ShapeDtypeStruct((_H, _H, _K), jnp.int32),
        jax.ShapeDtypeStruct((_H, _H, _K), jnp.float32),
        jax.ShapeDtypeStruct((_H, 1, 1), jnp.float32),
        jax.ShapeDtypeStruct((_H, 1, 1), jnp.float32),
    ]
    return pl.pallas_call(
        _nbr_body,
        grid=(_H,),
        in_specs=[
            pl.BlockSpec(memory_space=pltpu.SMEM),
            pl.BlockSpec(memory_space=pltpu.SMEM),
            pl.BlockSpec(memory_space=pltpu.SMEM),
            pl.BlockSpec(memory_space=pltpu.SMEM),
            pl.BlockSpec((_H, 1), lambda r: (0, 0)),
            pl.BlockSpec((1, _C), lambda r: (0, 0)),
            pl.BlockSpec((2, _C), lambda r: (0, 0)),
            pl.BlockSpec((2, _C), lambda r: (0, 0)),
        ],
        out_specs=[
            pl.BlockSpec((1, _H, _K), lambda r: (r, 0, 0)),
            pl.BlockSpec((1, _H, _K), lambda r: (r, 0, 0)),
            pl.BlockSpec((1, 1, 1), lambda r: (r, 0, 0)),
            pl.BlockSpec((1, 1, 1), lambda r: (r, 0, 0)),
        ],
        out_shape=out_shapes,
    )(lin, knots, S_m_x, S_m_y, lin.reshape(_H, 1),
      phi_w.reshape(1, _C), phi_i.T, phi_j.T)


_G = 16
_NW = 32
_DPW = (2 * _N) // _NW
_NCH = _DPW // _G


def _agg_body(x_hbm, gidx_hbm, w_hbm, out_hbm, idx_v, rows_v, w_v, out_v, sem):
    cid = lax.axis_index("c")
    sid = lax.axis_index("s")
    wid = sid * 2 + cid
    pltpu.sync_copy(gidx_hbm.at[pl.ds(wid * (_DPW * _K // 128), _DPW * _K // 128), :],
                    idx_v)
    pltpu.sync_copy(w_hbm.at[pl.ds(wid * _DPW, _DPW), :], w_v)

    def chunk(c, carry):
        d0 = wid * _DPW + c * _G
        handles = [
            pltpu.async_copy(x_hbm.at[idx_v.at[c * 4 + j]],
                             rows_v.at[pl.ds(j * 128, 128), :], sem)
            for j in range(4)
        ]
        for h in handles:
            h.wait()

        def g_body(g, carry2):
            accs = [jnp.zeros((16,), jnp.float32) for _ in range(8)]
            wrow = c * _G + g
            whalf = (w_v[wrow, pl.ds(0, 16)], w_v[wrow, pl.ds(16, 16)])
            for k in range(_K):
                wb = lax.gather(
                    whalf[k // 16],
                    jnp.full((16, 1), k % 16, jnp.int32),
                    lax.GatherDimensionNumbers(
                        offset_dims=(), collapsed_slice_dims=(0,),
                        start_index_map=(0,)),
                    (1,),
                    mode=lax.GatherScatterMode.PROMISE_IN_BOUNDS)
                row = g * _K + k
                for c8 in range(8):
                    accs[c8] = accs[c8] + wb * rows_v[row, pl.ds(c8 * 16, 16)]
            for c8 in range(8):
                out_v[g, pl.ds(c8 * 16, 16)] = accs[c8]
            return carry2

        lax.fori_loop(0, _G, g_body, 0)
        pltpu.sync_copy(out_v, out_hbm.at[pl.ds(d0, _G), :])
        return carry

    lax.fori_loop(0, _NCH, chunk, 0)


def _aggregate(x_flat, gidx2d, w2):
    mesh = plsc.VectorSubcoreMesh(core_axis_name="c", subcore_axis_name="s")
    kern = functools.partial(
        pl.kernel,
        mesh=mesh,
        out_type=jax.ShapeDtypeStruct((2 * _N, _C), jnp.float32),
        scratch_types=[
            pltpu.VMEM((_DPW * _K // 128, 128), jnp.int32),
            pltpu.VMEM((_G * _K, _C), jnp.float32),
            pltpu.VMEM((_DPW, _K), jnp.float32),
            pltpu.VMEM((_G, _C), jnp.float32),
            pltpu.SemaphoreType.DMA,
        ],
    )(_agg_body)
    return kern(x_flat, gidx2d, w2)


def _mlp_body(scale_smem, x_ref, w1t_ref, b1_ref, w2t_ref, b2_ref, agg_ref,
              o_ref):
    h = jnp.dot(x_ref[...], w1t_ref[...], preferred_element_type=jnp.float32)
    h = jnp.maximum(h + b1_ref[...], 0.0)
    o = jnp.dot(h, w2t_ref[...], preferred_element_type=jnp.float32)
    o_ref[...] = o + b2_ref[...] + scale_smem[0] * agg_ref[...]


def _mlp_combine(x_flat, W1T, W1_b, W2T, W2_b, agg, scale):
    rows = 2 * _N
    br = 512
    return pl.pallas_call(
        _mlp_body,
        grid=(rows // br,),
        in_specs=[
            pl.BlockSpec(memory_space=pltpu.SMEM),
            pl.BlockSpec((br, _C), lambda r: (r, 0)),
            pl.BlockSpec((_C, 2 * _C), lambda r: (0, 0)),
            pl.BlockSpec((1, 2 * _C), lambda r: (0, 0)),
            pl.BlockSpec((2 * _C, _C), lambda r: (0, 0)),
            pl.BlockSpec((1, _C), lambda r: (0, 0)),
            pl.BlockSpec((br, _C), lambda r: (r, 0)),
        ],
        out_specs=pl.BlockSpec((br, _C), lambda r: (r, 0)),
        out_shape=jax.ShapeDtypeStruct((rows, _C), jnp.float32),
    )(scale, x_flat, W1T, W1_b.reshape(1, 2 * _C), W2T, W2_b.reshape(1, _C),
      agg)


def kernel(x, W1_w, W1_b, W2_w, W2_b, phi_w, phi_i, phi_j,
           h1_w, h1_b, h2_w, h2_b, S_m_x, S_m_y):
    lin = jnp.linspace(0.0, 1.0, _H).astype(jnp.float32)
    knots = jnp.linspace(0.0, 1.0, _NKNOTS).astype(jnp.float32)
    jout, prod, psis, phis = _neighbors_and_weights(
        lin, knots, S_m_x, S_m_y, phi_w, phi_i, phi_j)
    jflat = jout.reshape(_N, _K)
    prod_flat = prod.reshape(_N, _K)
    mpsi = jnp.sum(psis) / (_N * _K)
    mphi = jnp.sum(phis) / (_N * _K)
    scale = 1.0 / ((mpsi + 1e-6) * (mphi + 1e-6) * jnp.float32(_K))
    gidx = jnp.concatenate([jflat, jflat + _N], axis=0)
    gidx = gidx.reshape((2 * _N * _K) // 128, 128)
    w2 = jnp.concatenate([prod_flat, prod_flat], axis=0)
    x_flat = x.reshape(2 * _N, _C)
    agg = _aggregate(x_flat, gidx, w2)
    out = _mlp_combine(x_flat, W1_w.T, W1_b, W2_w.T, W2_b, agg,
                       scale.reshape(1))
    return out.reshape(2, _N, _C)


# speedup vs baseline: 1.3015x; 1.3015x over previous
import functools

import jax
import jax.numpy as jnp
from jax import lax
from jax.experimental import pallas as pl
from jax.experimental.pallas import tpu as pltpu
from jax.experimental.pallas import tpu_sc as plsc

_N = 4096
_K = 32
_H = 64
_C = 128
_NKNOTS = 32


def _rne(a):
    b = lax.bitcast_convert_type(a, jnp.int32)
    r = (b + jnp.int32(0x7FFF) + ((b >> 16) & 1)) & jnp.int32(-65536)
    return lax.bitcast_convert_type(r, jnp.float32)


def _nbr_body(lin_smem, knots_smem, smx_smem, smy_smem,
              lin_col, lin_ext, phiw, phii, phij,
              jout, prod_out, psis_out, phis_out):
    r0 = pl.program_id(0)
    lin = lin_col[...]
    oiota = lax.broadcasted_iota(jnp.int32, (_H, 128), 1)
    ciota = lax.broadcasted_iota(jnp.int32, (_H, 128), 0)
    odr = oiota // 11 - 5
    odc = oiota % 11 - 5
    lin_sh = jnp.concatenate([lin_ext[s:s + _H, :] for s in range(11)], axis=1)
    linj = jnp.zeros((_H, 128), jnp.float32)
    for s in range(11):
        linj = jnp.where(odc == s - 5, lin_sh[:, s:s + 1], linj)
    dy = lin - linj
    xi = lin_smem[r0]
    dxv = jnp.zeros((1, 128), jnp.float32)
    for t in range(11):
        rj = jnp.clip(r0 + t - 5, 0, _H - 1)
        dxv = jnp.where(odr[0:1, :] == t - 5, xi - lin_smem[rj], dxv)
    dist = jnp.sqrt(dxv * dxv + dy * dy)
    cc = ciota + odc
    valid = ((cc >= 0) & (cc < _H) & (r0 + odr >= 0) & (r0 + odr < _H)
             & (oiota < 121))
    dist = jnp.where(valid, dist, jnp.inf)
    liota_f = oiota.astype(jnp.float32)
    sel = []
    for _ in range(_K):
        m = jnp.min(dist, axis=1, keepdims=True)
        cand = jnp.where(dist == m, liota_f, jnp.float32(1e9))
        amin = jnp.min(cand, axis=1, keepdims=True)
        sel.append(amin)
        dist = jnp.where(cand == amin, jnp.inf, dist)
    osel = jnp.concatenate(sel, axis=1).astype(jnp.int32)
    drsel = osel // 11 - 5
    dcsel = osel % 11 - 5
    ci_col = lax.broadcasted_iota(jnp.int32, (_H, 1), 0)
    jout[0] = (r0 + drsel) * _H + ci_col + dcsel
    cjx = jnp.zeros((_H, _K), jnp.float32)
    cjy = jnp.zeros((_H, _K), jnp.float32)
    for t in range(11):
        rj = jnp.clip(r0 + t - 5, 0, _H - 1)
        cjx = jnp.where(drsel == t - 5, lin_smem[rj], cjx)
    for s in range(11):
        cjy = jnp.where(dcsel == s - 5, lin_sh[:, s:s + 1], cjy)
    relx = xi - cjx
    rely = lin - cjy
    psi_x = jnp.zeros((_H, _K), jnp.float32)
    psi_y = jnp.zeros((_H, _K), jnp.float32)
    for t in range(_NKNOTS):
        kv = knots_smem[t]
        psi_x = psi_x + jnp.maximum(1.0 - jnp.abs(relx - kv), 0.0) * smx_smem[t]
        psi_y = psi_y + jnp.maximum(1.0 - jnp.abs(rely - kv), 0.0) * smy_smem[t]
    psi = psi_x * psi_y
    rpw = _rne(phiw[...])
    rpi = _rne(phii[...])
    rpj = _rne(phij[...])
    i_e = _rne(xi) * rpi[0:1, :] + _rne(lin) * rpi[1:2, :]
    j_e = (_rne(cjx)[:, :, None] * rpj[0:1, :].reshape(1, 1, _C)
           + _rne(cjy)[:, :, None] * rpj[1:2, :].reshape(1, 1, _C))
    t = i_e[:, None, :] * j_e
    phiv = jnp.sum(_rne(t) * rpw.reshape(1, 1, _C), axis=2)
    prod_out[0] = psi * phiv
    psis_out[0] = jnp.sum(jnp.abs(psi)).reshape(1, 1)
    phis_out[0] = jnp.sum(jnp.abs(phiv)).reshape(1, 1)


def _neighbors_and_weights(lin, knots, S_m_x, S_m_y, phi_w, phi_i, phi_j):
    out_shapes = [
        jax.ShapeDtypeStruct((_H, _H, _K), jnp.int32),
        jax.ShapeDtypeStruct((_H, _H, _K), jnp.float32),
        jax.ShapeDtypeStruct((_H, 1, 1), jnp.float32),
        jax.ShapeDtypeStruct((_H, 1, 1), jnp.float32),
    ]
    call = pl.pallas_call(
        _nbr_body,
        grid=(_H,),
        in_specs=[
            pl.BlockSpec(memory_space=pltpu.SMEM),
            pl.BlockSpec(memory_space=pltpu.SMEM),
            pl.BlockSpec(memory_space=pltpu.SMEM),
            pl.BlockSpec(memory_space=pltpu.SMEM),
            pl.BlockSpec((_H, 1), lambda r: (0, 0)),
            pl.BlockSpec((80, 1), lambda r: (0, 0)),
            pl.BlockSpec((1, _C), lambda r: (0, 0)),
            pl.BlockSpec((2, _C), lambda r: (0, 0)),
            pl.BlockSpec((2, _C), lambda r: (0, 0)),
        ],
        out_specs=[
            pl.BlockSpec((1, _H, _K), lambda r: (r, 0, 0)),
            pl.BlockSpec((1, _H, _K), lambda r: (r, 0, 0)),
            pl.BlockSpec((1, 1, 1), lambda r: (r, 0, 0)),
            pl.BlockSpec((1, 1, 1), lambda r: (r, 0, 0)),
        ],
        out_shape=out_shapes,
    )
    lin_ext = jnp.concatenate(
        [jnp.zeros((5,), jnp.float32), lin, jnp.zeros((11,), jnp.float32)])
    return call(lin, knots, S_m_x, S_m_y, lin.reshape(_H, 1),
                lin_ext.reshape(80, 1), phi_w.reshape(1, _C), phi_i.T,
                phi_j.T)


_G = 16
_NW = 32
_DPW = (2 * _N) // _NW
_NCH = _DPW // _G


def _agg_body(x_hbm, gidx_hbm, w_hbm, out_hbm, idx_v, rows_v, w_v, out_v, sem):
    cid = lax.axis_index("c")
    sid = lax.axis_index("s")
    wid = sid * 2 + cid
    pltpu.sync_copy(gidx_hbm.at[pl.ds(wid * (_DPW * _K // 128), _DPW * _K // 128), :],
                    idx_v)
    pltpu.sync_copy(w_hbm.at[pl.ds(wid * _DPW, _DPW), :], w_v)

    def chunk(c, carry):
        d0 = wid * _DPW + c * _G
        handles = [
            pltpu.async_copy(x_hbm.at[idx_v.at[c * 4 + j]],
                             rows_v.at[pl.ds(j * 128, 128), :], sem)
            for j in range(4)
        ]
        for h in handles:
            h.wait()

        def g_body(g, carry2):
            accs = [jnp.zeros((16,), jnp.float32) for _ in range(8)]
            wrow = c * _G + g
            whalf = (w_v[wrow, pl.ds(0, 16)], w_v[wrow, pl.ds(16, 16)])
            for k in range(_K):
                wb = lax.gather(
                    whalf[k // 16],
                    jnp.full((16, 1), k % 16, jnp.int32),
                    lax.GatherDimensionNumbers(
                        offset_dims=(), collapsed_slice_dims=(0,),
                        start_index_map=(0,)),
                    (1,),
                    mode=lax.GatherScatterMode.PROMISE_IN_BOUNDS)
                row = g * _K + k
                for c8 in range(8):
                    accs[c8] = accs[c8] + wb * rows_v[row, pl.ds(c8 * 16, 16)]
            for c8 in range(8):
                out_v[g, pl.ds(c8 * 16, 16)] = accs[c8]
            return carry2

        lax.fori_loop(0, _G, g_body, 0)
        pltpu.sync_copy(out_v, out_hbm.at[pl.ds(d0, _G), :])
        return carry

    lax.fori_loop(0, _NCH, chunk, 0)


def _aggregate(x_flat, gidx2d, w2):
    mesh = plsc.VectorSubcoreMesh(core_axis_name="c", subcore_axis_name="s")
    kern = functools.partial(
        pl.kernel,
        mesh=mesh,
        out_type=jax.ShapeDtypeStruct((2 * _N, _C), jnp.float32),
        scratch_types=[
            pltpu.VMEM((_DPW * _K // 128, 128), jnp.int32),
            pltpu.VMEM((_G * _K, _C), jnp.float32),
            pltpu.VMEM((_DPW, _K), jnp.float32),
            pltpu.VMEM((_G, _C), jnp.float32),
            pltpu.SemaphoreType.DMA,
        ],
    )(_agg_body)
    return kern(x_flat, gidx2d, w2)


def _mlp_body(scale_smem, x_ref, w1t_ref, b1_ref, w2t_ref, b2_ref, agg_ref,
              o_ref):
    h = jnp.dot(x_ref[...], w1t_ref[...], preferred_element_type=jnp.float32)
    h = jnp.maximum(h + b1_ref[...], 0.0)
    o = jnp.dot(h, w2t_ref[...], preferred_element_type=jnp.float32)
    o_ref[...] = o + b2_ref[...] + scale_smem[0] * agg_ref[...]


def _mlp_combine(x_flat, W1T, W1_b, W2T, W2_b, agg, scale):
    rows = 2 * _N
    br = 512
    return pl.pallas_call(
        _mlp_body,
        grid=(rows // br,),
        in_specs=[
            pl.BlockSpec(memory_space=pltpu.SMEM),
            pl.BlockSpec((br, _C), lambda r: (r, 0)),
            pl.BlockSpec((_C, 2 * _C), lambda r: (0, 0)),
            pl.BlockSpec((1, 2 * _C), lambda r: (0, 0)),
            pl.BlockSpec((2 * _C, _C), lambda r: (0, 0)),
            pl.BlockSpec((1, _C), lambda r: (0, 0)),
            pl.BlockSpec((br, _C), lambda r: (r, 0)),
        ],
        out_specs=pl.BlockSpec((br, _C), lambda r: (r, 0)),
        out_shape=jax.ShapeDtypeStruct((rows, _C), jnp.float32),
    )(scale, x_flat, W1T, W1_b.reshape(1, 2 * _C), W2T, W2_b.reshape(1, _C),
      agg)


def kernel(x, W1_w, W1_b, W2_w, W2_b, phi_w, phi_i, phi_j,
           h1_w, h1_b, h2_w, h2_b, S_m_x, S_m_y):
    lin = jnp.linspace(0.0, 1.0, _H).astype(jnp.float32)
    knots = jnp.linspace(0.0, 1.0, _NKNOTS).astype(jnp.float32)
    jout, prod, psis, phis = _neighbors_and_weights(
        lin, knots, S_m_x, S_m_y, phi_w, phi_i, phi_j)
    jflat = jout.reshape(_N, _K)
    prod_flat = prod.reshape(_N, _K)
    mpsi = jnp.sum(psis) / (_N * _K)
    mphi = jnp.sum(phis) / (_N * _K)
    scale = 1.0 / ((mpsi + 1e-6) * (mphi + 1e-6) * jnp.float32(_K))
    gidx = jnp.concatenate([jflat, jflat + _N], axis=0)
    gidx = gidx.reshape((2 * _N * _K) // 128, 128)
    w2 = jnp.concatenate([prod_flat, prod_flat], axis=0)
    x_flat = x.reshape(2 * _N, _C)
    agg = _aggregate(x_flat, gidx, w2)
    out = _mlp_combine(x_flat, W1_w.T, W1_b, W2_w.T, W2_b, agg,
                       scale.reshape(1))
    return out.reshape(2, _N, _C)


# speedup vs baseline: 1.4156x; 1.0877x over previous
import functools

import jax
import jax.numpy as jnp
from jax import lax
from jax.experimental import pallas as pl
from jax.experimental.pallas import tpu as pltpu
from jax.experimental.pallas import tpu_sc as plsc

_N = 4096
_K = 32
_H = 64
_C = 128
_NKNOTS = 32


def _rne(a):
    b = lax.bitcast_convert_type(a, jnp.int32)
    r = (b + jnp.int32(0x7FFF) + ((b >> 16) & 1)) & jnp.int32(-65536)
    return lax.bitcast_convert_type(r, jnp.float32)


def _nbr_body(lin_smem, knots_smem, smx_smem, smy_smem,
              lin_col, lin_ext, phiw, phii, phij,
              jout, prod_out, psis_out, phis_out):
    r0 = pl.program_id(0)
    lin = lin_col[...]
    oiota = lax.broadcasted_iota(jnp.int32, (_H, 128), 1)
    ciota = lax.broadcasted_iota(jnp.int32, (_H, 128), 0)
    odr = oiota // 11 - 5
    odc = oiota % 11 - 5
    lin_sh = jnp.concatenate([lin_ext[s:s + _H, :] for s in range(11)], axis=1)
    linj = jnp.zeros((_H, 128), jnp.float32)
    for s in range(11):
        linj = jnp.where(odc == s - 5, lin_sh[:, s:s + 1], linj)
    dy = lin - linj
    xi = lin_smem[r0]
    dxv = jnp.zeros((1, 128), jnp.float32)
    for t in range(11):
        rj = jnp.clip(r0 + t - 5, 0, _H - 1)
        dxv = jnp.where(odr[0:1, :] == t - 5, xi - lin_smem[rj], dxv)
    dist = jnp.sqrt(dxv * dxv + dy * dy)
    cc = ciota + odc
    valid = ((cc >= 0) & (cc < _H) & (r0 + odr >= 0) & (r0 + odr < _H)
             & (oiota < 121))
    dist = jnp.where(valid, dist, jnp.inf)
    liota_f = oiota.astype(jnp.float32)
    sel = []
    for _ in range(_K):
        m = jnp.min(dist, axis=1, keepdims=True)
        cand = jnp.where(dist == m, liota_f, jnp.float32(1e9))
        amin = jnp.min(cand, axis=1, keepdims=True)
        sel.append(amin)
        dist = jnp.where(cand == amin, jnp.inf, dist)
    osel = jnp.concatenate(sel, axis=1).astype(jnp.int32)
    drsel = osel // 11 - 5
    dcsel = osel % 11 - 5
    ci_col = lax.broadcasted_iota(jnp.int32, (_H, 1), 0)
    jout[0] = (r0 + drsel) * _H + ci_col + dcsel
    cjx = jnp.zeros((_H, _K), jnp.float32)
    cjy = jnp.zeros((_H, _K), jnp.float32)
    for t in range(11):
        rj = jnp.clip(r0 + t - 5, 0, _H - 1)
        cjx = jnp.where(drsel == t - 5, lin_smem[rj], cjx)
    for s in range(11):
        cjy = jnp.where(dcsel == s - 5, lin_sh[:, s:s + 1], cjy)
    relx = xi - cjx
    rely = lin - cjy
    relcat = jnp.concatenate([relx, rely], axis=1)
    kiota = lax.broadcasted_iota(jnp.int32, (1, 2 * _K), 1)
    psic = jnp.zeros((_H, 2 * _K), jnp.float32)
    for t in range(_NKNOTS):
        kv = knots_smem[t]
        wrow = jnp.where(kiota < _K, smx_smem[t], smy_smem[t])
        psic = psic + jnp.maximum(1.0 - jnp.abs(relcat - kv), 0.0) * wrow
    psi = psic[:, 0:_K] * psic[:, _K:2 * _K]
    rpw = _rne(phiw[...])
    rpi = _rne(phii[...])
    rpj = _rne(phij[...])
    i_e = _rne(xi) * rpi[0:1, :] + _rne(lin) * rpi[1:2, :]
    j_e = (_rne(cjx)[:, :, None] * rpj[0:1, :].reshape(1, 1, _C)
           + _rne(cjy)[:, :, None] * rpj[1:2, :].reshape(1, 1, _C))
    t = i_e[:, None, :] * j_e
    phiv = jnp.sum(_rne(t) * rpw.reshape(1, 1, _C), axis=2)
    prod_out[0] = psi * phiv
    psis_out[0] = jnp.sum(jnp.abs(psi)).reshape(1, 1)
    phis_out[0] = jnp.sum(jnp.abs(phiv)).reshape(1, 1)


def _neighbors_and_weights(lin, knots, S_m_x, S_m_y, phi_w, phi_i, phi_j):
    out_shapes = [
        jax.ShapeDtypeStruct((_H, _H, _K), jnp.int32),
        jax.ShapeDtypeStruct((_H, _H, _K), jnp.float32),
        jax.ShapeDtypeStruct((_H, 1, 1), jnp.float32),
        jax.ShapeDtypeStruct((_H, 1, 1), jnp.float32),
    ]
    call = pl.pallas_call(
        _nbr_body,
        grid=(_H,),
        in_specs=[
            pl.BlockSpec(memory_space=pltpu.SMEM),
            pl.BlockSpec(memory_space=pltpu.SMEM),
            pl.BlockSpec(memory_space=pltpu.SMEM),
            pl.BlockSpec(memory_space=pltpu.SMEM),
            pl.BlockSpec((_H, 1), lambda r: (0, 0)),
            pl.BlockSpec((80, 1), lambda r: (0, 0)),
            pl.BlockSpec((1, _C), lambda r: (0, 0)),
            pl.BlockSpec((2, _C), lambda r: (0, 0)),
            pl.BlockSpec((2, _C), lambda r: (0, 0)),
        ],
        out_specs=[
            pl.BlockSpec((1, _H, _K), lambda r: (r, 0, 0)),
            pl.BlockSpec((1, _H, _K), lambda r: (r, 0, 0)),
            pl.BlockSpec((1, 1, 1), lambda r: (r, 0, 0)),
            pl.BlockSpec((1, 1, 1), lambda r: (r, 0, 0)),
        ],
        out_shape=out_shapes,
    )
    lin_ext = jnp.concatenate(
        [jnp.zeros((5,), jnp.float32), lin, jnp.zeros((11,), jnp.float32)])
    return call(lin, knots, S_m_x, S_m_y, lin.reshape(_H, 1),
                lin_ext.reshape(80, 1), phi_w.reshape(1, _C), phi_i.T,
                phi_j.T)


_G = 8
_NW = 32
_DPW = (2 * _N) // _NW
_NCH = _DPW // _G


def _agg_body(x_hbm, gidx_hbm, w_hbm, out_hbm,
              idx_v, w_v, rows0, rows1, out0, out1, sem0, sem1):
    cid = lax.axis_index("c")
    sid = lax.axis_index("s")
    wid = sid * 2 + cid
    pltpu.sync_copy(gidx_hbm.at[pl.ds(wid * (_DPW * _K // 128), _DPW * _K // 128), :],
                    idx_v)
    pltpu.sync_copy(w_hbm.at[pl.ds(wid * _DPW, _DPW), :], w_v)

    def issue(c, rows, sem):
        for j in range(2):
            pltpu.async_copy(x_hbm.at[idx_v.at[c * 2 + j]],
                             rows.at[pl.ds(j * 128, 128), :], sem)

    def drain(rows, sem):
        for j in range(2):
            pltpu.make_async_copy(x_hbm.at[pl.ds(0, 128), :],
                                  rows.at[pl.ds(j * 128, 128), :], sem).wait()

    def compute(c, rows, outv):
        def g_body(g, carry2):
            accs = [jnp.zeros((16,), jnp.float32) for _ in range(8)]
            wrow = c * _G + g
            whalf = (w_v[wrow, pl.ds(0, 16)], w_v[wrow, pl.ds(16, 16)])
            for k in range(_K):
                wb = lax.gather(
                    whalf[k // 16],
                    jnp.full((16, 1), k % 16, jnp.int32),
                    lax.GatherDimensionNumbers(
                        offset_dims=(), collapsed_slice_dims=(0,),
                        start_index_map=(0,)),
                    (1,),
                    mode=lax.GatherScatterMode.PROMISE_IN_BOUNDS)
                row = g * _K + k
                for c8 in range(8):
                    accs[c8] = accs[c8] + wb * rows[row, pl.ds(c8 * 16, 16)]
            for c8 in range(8):
                outv[g, pl.ds(c8 * 16, 16)] = accs[c8]
            return carry2

        lax.fori_loop(0, _G, g_body, 0)
        pltpu.sync_copy(outv, out_hbm.at[pl.ds(wid * _DPW + c * _G, _G), :])

    issue(0, rows0, sem0)

    def pair(i, carry):
        c0 = 2 * i
        issue(c0 + 1, rows1, sem1)
        drain(rows0, sem0)
        compute(c0, rows0, out0)

        @pl.when(i < _NCH // 2 - 1)
        def _():
            issue(c0 + 2, rows0, sem0)

        drain(rows1, sem1)
        compute(c0 + 1, rows1, out1)
        return carry

    lax.fori_loop(0, _NCH // 2, pair, 0)


def _aggregate(x_flat, gidx2d, w2):
    mesh = plsc.VectorSubcoreMesh(core_axis_name="c", subcore_axis_name="s")
    kern = functools.partial(
        pl.kernel,
        mesh=mesh,
        out_type=jax.ShapeDtypeStruct((2 * _N, _C), jnp.float32),
        scratch_types=[
            pltpu.VMEM((_DPW * _K // 128, 128), jnp.int32),
            pltpu.VMEM((_DPW, _K), jnp.float32),
            pltpu.VMEM((_G * _K, _C), jnp.float32),
            pltpu.VMEM((_G * _K, _C), jnp.float32),
            pltpu.VMEM((_G, _C), jnp.float32),
            pltpu.VMEM((_G, _C), jnp.float32),
            pltpu.SemaphoreType.DMA,
            pltpu.SemaphoreType.DMA,
        ],
    )(_agg_body)
    return kern(x_flat, gidx2d, w2)


def _mlp_body(scale_smem, x_ref, w1t_ref, b1_ref, w2t_ref, b2_ref, agg_ref,
              o_ref):
    h = jnp.dot(x_ref[...], w1t_ref[...], preferred_element_type=jnp.float32)
    h = jnp.maximum(h + b1_ref[...], 0.0)
    o = jnp.dot(h, w2t_ref[...], preferred_element_type=jnp.float32)
    o_ref[...] = o + b2_ref[...] + scale_smem[0] * agg_ref[...]


def _mlp_combine(x_flat, W1T, W1_b, W2T, W2_b, agg, scale):
    rows = 2 * _N
    br = 512
    return pl.pallas_call(
        _mlp_body,
        grid=(rows // br,),
        in_specs=[
            pl.BlockSpec(memory_space=pltpu.SMEM),
            pl.BlockSpec((br, _C), lambda r: (r, 0)),
            pl.BlockSpec((_C, 2 * _C), lambda r: (0, 0)),
            pl.BlockSpec((1, 2 * _C), lambda r: (0, 0)),
            pl.BlockSpec((2 * _C, _C), lambda r: (0, 0)),
            pl.BlockSpec((1, _C), lambda r: (0, 0)),
            pl.BlockSpec((br, _C), lambda r: (r, 0)),
        ],
        out_specs=pl.BlockSpec((br, _C), lambda r: (r, 0)),
        out_shape=jax.ShapeDtypeStruct((rows, _C), jnp.float32),
    )(scale, x_flat, W1T, W1_b.reshape(1, 2 * _C), W2T, W2_b.reshape(1, _C),
      agg)


def kernel(x, W1_w, W1_b, W2_w, W2_b, phi_w, phi_i, phi_j,
           h1_w, h1_b, h2_w, h2_b, S_m_x, S_m_y):
    lin = jnp.linspace(0.0, 1.0, _H).astype(jnp.float32)
    knots = jnp.linspace(0.0, 1.0, _NKNOTS).astype(jnp.float32)
    jout, prod, psis, phis = _neighbors_and_weights(
        lin, knots, S_m_x, S_m_y, phi_w, phi_i, phi_j)
    jflat = jout.reshape(_N, _K)
    prod_flat = prod.reshape(_N, _K)
    mpsi = jnp.sum(psis) / (_N * _K)
    mphi = jnp.sum(phis) / (_N * _K)
    scale = 1.0 / ((mpsi + 1e-6) * (mphi + 1e-6) * jnp.float32(_K))
    gidx = jnp.concatenate([jflat, jflat + _N], axis=0)
    gidx = gidx.reshape((2 * _N * _K) // 128, 128)
    w2 = jnp.concatenate([prod_flat, prod_flat], axis=0)
    x_flat = x.reshape(2 * _N, _C)
    agg = _aggregate(x_flat, gidx, w2)
    out = _mlp_combine(x_flat, W1_w.T, W1_b, W2_w.T, W2_b, agg,
                       scale.reshape(1))
    return out.reshape(2, _N, _C)


# speedup vs baseline: 2.0969x; 1.4812x over previous
import functools

import jax
import jax.numpy as jnp
from jax import lax
from jax.experimental import pallas as pl
from jax.experimental.pallas import tpu as pltpu
from jax.experimental.pallas import tpu_sc as plsc

_N = 4096
_K = 32
_H = 64
_C = 128
_NKNOTS = 32


def _rne(a):
    b = lax.bitcast_convert_type(a, jnp.int32)
    r = (b + jnp.int32(0x7FFF) + ((b >> 16) & 1)) & jnp.int32(-65536)
    return lax.bitcast_convert_type(r, jnp.float32)


_P = 128


def _nbr_body(lin_smem, knots_smem, smx_smem, smy_smem,
              lin2_col, lin_ext, phiw, phii, phij,
              jout, prod_out, psis_out, phis_out):
    r0 = pl.program_id(0) * 2
    lin2 = lin2_col[...]
    oiota = lax.broadcasted_iota(jnp.int32, (_P, 128), 1)
    iiota = lax.broadcasted_iota(jnp.int32, (_P, 128), 0)
    odr = oiota // 11 - 5
    odc = oiota % 11 - 5
    ccol = iiota % _H
    qtop = iiota < _H
    lsh1 = jnp.concatenate([lin_ext[s:s + _H, :] for s in range(11)], axis=1)
    lin_sh = jnp.concatenate([lsh1, lsh1], axis=0)
    linj = jnp.zeros((_P, 128), jnp.float32)
    for s in range(11):
        linj = jnp.where(odc == s - 5, lin_sh[:, s:s + 1], linj)
    dy = lin2 - linj
    xi0 = lin_smem[r0]
    xi1 = lin_smem[r0 + 1]
    qtop_c = qtop[:, 0:1]
    xi_col = jnp.where(qtop_c, xi0, xi1)
    dxv = jnp.zeros((_P, 128), jnp.float32)
    for t in range(11):
        rj0 = jnp.clip(r0 + t - 5, 0, _H - 1)
        rj1 = jnp.clip(r0 + 1 + t - 5, 0, _H - 1)
        vcol = jnp.where(qtop_c, xi0 - lin_smem[rj0], xi1 - lin_smem[rj1])
        dxv = jnp.where(odr == t - 5, vcol, dxv)
    dist = jnp.sqrt(dxv * dxv + dy * dy)
    cc = ccol + odc
    rr = r0 + jnp.where(qtop, 0, 1) + odr
    valid = ((cc >= 0) & (cc < _H) & (rr >= 0) & (rr < _H) & (oiota < 121))
    dist = jnp.where(valid, dist, jnp.inf)
    liota_f = oiota.astype(jnp.float32)
    sel = []
    for _ in range(_K):
        m = jnp.min(dist, axis=1, keepdims=True)
        cand = jnp.where(dist == m, liota_f, jnp.float32(1e9))
        amin = jnp.min(cand, axis=1, keepdims=True)
        sel.append(amin)
        dist = jnp.where(cand == amin, jnp.inf, dist)
    osel = jnp.concatenate(sel, axis=1).astype(jnp.int32)
    drsel = osel // 11 - 5
    dcsel = osel % 11 - 5
    ii_col = lax.broadcasted_iota(jnp.int32, (_P, 1), 0)
    q_col = ii_col // _H
    c_col = ii_col % _H
    jout[0] = (r0 + q_col + drsel) * _H + c_col + dcsel
    cjx = jnp.zeros((_P, _K), jnp.float32)
    cjy = jnp.zeros((_P, _K), jnp.float32)
    for t in range(11):
        rj0 = jnp.clip(r0 + t - 5, 0, _H - 1)
        rj1 = jnp.clip(r0 + 1 + t - 5, 0, _H - 1)
        vcol = jnp.where(qtop_c, lin_smem[rj0], lin_smem[rj1])
        cjx = jnp.where(drsel == t - 5, vcol, cjx)
    for s in range(11):
        cjy = jnp.where(dcsel == s - 5, lin_sh[:, s:s + 1], cjy)
    relx = xi_col - cjx
    rely = lin2 - cjy
    relcat = jnp.concatenate([relx, rely], axis=1)
    kiota = lax.broadcasted_iota(jnp.int32, (1, 2 * _K), 1)
    psic = jnp.zeros((_P, 2 * _K), jnp.float32)
    for t in range(_NKNOTS):
        kv = knots_smem[t]
        wrow = jnp.where(kiota < _K, smx_smem[t], smy_smem[t])
        psic = psic + jnp.maximum(1.0 - jnp.abs(relcat - kv), 0.0) * wrow
    psi = psic[:, 0:_K] * psic[:, _K:2 * _K]
    rpw = _rne(phiw[...])
    rpi = _rne(phii[...])
    rpj = _rne(phij[...])
    i_e = _rne(xi_col) * rpi[0:1, :] + _rne(lin2) * rpi[1:2, :]
    j_e = (_rne(cjx)[:, :, None] * rpj[0:1, :].reshape(1, 1, _C)
           + _rne(cjy)[:, :, None] * rpj[1:2, :].reshape(1, 1, _C))
    t = i_e[:, None, :] * j_e
    phiv = jnp.sum(_rne(t) * rpw.reshape(1, 1, _C), axis=2)
    prod_out[0] = psi * phiv
    psis_out[0] = jnp.sum(jnp.abs(psi)).reshape(1, 1)
    phis_out[0] = jnp.sum(jnp.abs(phiv)).reshape(1, 1)


def _neighbors_and_weights(lin, knots, S_m_x, S_m_y, phi_w, phi_i, phi_j):
    nb = _H // 2
    out_shapes = [
        jax.ShapeDtypeStruct((nb, _P, _K), jnp.int32),
        jax.ShapeDtypeStruct((nb, _P, _K), jnp.float32),
        jax.ShapeDtypeStruct((nb, 1, 1), jnp.float32),
        jax.ShapeDtypeStruct((nb, 1, 1), jnp.float32),
    ]
    call = pl.pallas_call(
        _nbr_body,
        grid=(nb,),
        in_specs=[
            pl.BlockSpec(memory_space=pltpu.SMEM),
            pl.BlockSpec(memory_space=pltpu.SMEM),
            pl.BlockSpec(memory_space=pltpu.SMEM),
            pl.BlockSpec(memory_space=pltpu.SMEM),
            pl.BlockSpec((_P, 1), lambda r: (0, 0)),
            pl.BlockSpec((80, 1), lambda r: (0, 0)),
            pl.BlockSpec((1, _C), lambda r: (0, 0)),
            pl.BlockSpec((2, _C), lambda r: (0, 0)),
            pl.BlockSpec((2, _C), lambda r: (0, 0)),
        ],
        out_specs=[
            pl.BlockSpec((1, _P, _K), lambda r: (r, 0, 0)),
            pl.BlockSpec((1, _P, _K), lambda r: (r, 0, 0)),
            pl.BlockSpec((1, 1, 1), lambda r: (r, 0, 0)),
            pl.BlockSpec((1, 1, 1), lambda r: (r, 0, 0)),
        ],
        out_shape=out_shapes,
    )
    lin_ext = jnp.concatenate(
        [jnp.zeros((5,), jnp.float32), lin, jnp.zeros((11,), jnp.float32)])
    lin2 = jnp.concatenate([lin, lin]).reshape(_P, 1)
    return call(lin, knots, S_m_x, S_m_y, lin2,
                lin_ext.reshape(80, 1), phi_w.reshape(1, _C), phi_i.T,
                phi_j.T)


_G = 8
_NW = 32
_DPW = (2 * _N) // _NW
_NCH = _DPW // _G


def _agg_body(x_hbm, gidx_hbm, w_hbm, out_hbm,
              idx_v, w_v, rows0, rows1, out0, out1, sem0, sem1):
    cid = lax.axis_index("c")
    sid = lax.axis_index("s")
    wid = sid * 2 + cid
    pltpu.sync_copy(gidx_hbm.at[pl.ds(wid * (_DPW * _K // 128), _DPW * _K // 128), :],
                    idx_v)
    pltpu.sync_copy(w_hbm.at[pl.ds(wid * _DPW, _DPW), :], w_v)

    def issue(c, rows, sem):
        for j in range(2):
            pltpu.async_copy(x_hbm.at[idx_v.at[c * 2 + j]],
                             rows.at[pl.ds(j * 128, 128), :], sem)

    def drain(rows, sem):
        for j in range(2):
            pltpu.make_async_copy(x_hbm.at[pl.ds(0, 128), :],
                                  rows.at[pl.ds(j * 128, 128), :], sem).wait()

    def compute(c, rows, outv):
        def g_body(g, carry2):
            accs = [jnp.zeros((16,), jnp.float32) for _ in range(8)]
            wrow = c * _G + g
            whalf = (w_v[wrow, pl.ds(0, 16)], w_v[wrow, pl.ds(16, 16)])
            for k in range(_K):
                wb = lax.gather(
                    whalf[k // 16],
                    jnp.full((16, 1), k % 16, jnp.int32),
                    lax.GatherDimensionNumbers(
                        offset_dims=(), collapsed_slice_dims=(0,),
                        start_index_map=(0,)),
                    (1,),
                    mode=lax.GatherScatterMode.PROMISE_IN_BOUNDS)
                row = g * _K + k
                for c8 in range(8):
                    accs[c8] = accs[c8] + wb * rows[row, pl.ds(c8 * 16, 16)]
            for c8 in range(8):
                outv[g, pl.ds(c8 * 16, 16)] = accs[c8]
            return carry2

        lax.fori_loop(0, _G, g_body, 0)
        pltpu.sync_copy(outv, out_hbm.at[pl.ds(wid * _DPW + c * _G, _G), :])

    issue(0, rows0, sem0)

    def pair(i, carry):
        c0 = 2 * i
        issue(c0 + 1, rows1, sem1)
        drain(rows0, sem0)
        compute(c0, rows0, out0)

        @pl.when(i < _NCH // 2 - 1)
        def _():
            issue(c0 + 2, rows0, sem0)

        drain(rows1, sem1)
        compute(c0 + 1, rows1, out1)
        return carry

    lax.fori_loop(0, _NCH // 2, pair, 0)


def _aggregate(x_flat, gidx2d, w2):
    mesh = plsc.VectorSubcoreMesh(core_axis_name="c", subcore_axis_name="s")
    kern = functools.partial(
        pl.kernel,
        mesh=mesh,
        out_type=jax.ShapeDtypeStruct((2 * _N, _C), jnp.float32),
        scratch_types=[
            pltpu.VMEM((_DPW * _K // 128, 128), jnp.int32),
            pltpu.VMEM((_DPW, _K), jnp.float32),
            pltpu.VMEM((_G * _K, _C), jnp.float32),
            pltpu.VMEM((_G * _K, _C), jnp.float32),
            pltpu.VMEM((_G, _C), jnp.float32),
            pltpu.VMEM((_G, _C), jnp.float32),
            pltpu.SemaphoreType.DMA,
            pltpu.SemaphoreType.DMA,
        ],
    )(_agg_body)
    return kern(x_flat, gidx2d, w2)


def _mlp_body(scale_smem, x_ref, w1t_ref, b1_ref, w2t_ref, b2_ref, agg_ref,
              o_ref):
    h = jnp.dot(x_ref[...], w1t_ref[...], preferred_element_type=jnp.float32)
    h = jnp.maximum(h + b1_ref[...], 0.0)
    o = jnp.dot(h, w2t_ref[...], preferred_element_type=jnp.float32)
    o_ref[...] = o + b2_ref[...] + scale_smem[0] * agg_ref[...]


def _mlp_combine(x_flat, W1T, W1_b, W2T, W2_b, agg, scale):
    rows = 2 * _N
    br = 512
    return pl.pallas_call(
        _mlp_body,
        grid=(rows // br,),
        in_specs=[
            pl.BlockSpec(memory_space=pltpu.SMEM),
            pl.BlockSpec((br, _C), lambda r: (r, 0)),
            pl.BlockSpec((_C, 2 * _C), lambda r: (0, 0)),
            pl.BlockSpec((1, 2 * _C), lambda r: (0, 0)),
            pl.BlockSpec((2 * _C, _C), lambda r: (0, 0)),
            pl.BlockSpec((1, _C), lambda r: (0, 0)),
            pl.BlockSpec((br, _C), lambda r: (r, 0)),
        ],
        out_specs=pl.BlockSpec((br, _C), lambda r: (r, 0)),
        out_shape=jax.ShapeDtypeStruct((rows, _C), jnp.float32),
    )(scale, x_flat, W1T, W1_b.reshape(1, 2 * _C), W2T, W2_b.reshape(1, _C),
      agg)


def kernel(x, W1_w, W1_b, W2_w, W2_b, phi_w, phi_i, phi_j,
           h1_w, h1_b, h2_w, h2_b, S_m_x, S_m_y):
    lin = jnp.linspace(0.0, 1.0, _H).astype(jnp.float32)
    knots = jnp.linspace(0.0, 1.0, _NKNOTS).astype(jnp.float32)
    jout, prod, psis, phis = _neighbors_and_weights(
        lin, knots, S_m_x, S_m_y, phi_w, phi_i, phi_j)
    jflat = jout.reshape(_N, _K)
    prod_flat = prod.reshape(_N, _K)
    mpsi = jnp.sum(psis) / (_N * _K)
    mphi = jnp.sum(phis) / (_N * _K)
    scale = 1.0 / ((mpsi + 1e-6) * (mphi + 1e-6) * jnp.float32(_K))
    gidx = jnp.concatenate([jflat, jflat + _N], axis=0)
    gidx = gidx.reshape((2 * _N * _K) // 128, 128)
    w2 = jnp.concatenate([prod_flat, prod_flat], axis=0)
    x_flat = x.reshape(2 * _N, _C)
    agg = _aggregate(x_flat, gidx, w2)
    out = _mlp_combine(x_flat, W1_w.T, W1_b, W2_w.T, W2_b, agg,
                       scale.reshape(1))
    return out.reshape(2, _N, _C)


# speedup vs baseline: 2.7358x; 1.3047x over previous
import functools

import jax
import jax.numpy as jnp
from jax import lax
from jax.experimental import pallas as pl
from jax.experimental.pallas import tpu as pltpu
from jax.experimental.pallas import tpu_sc as plsc

_N = 4096
_K = 32
_H = 64
_C = 128
_NKNOTS = 32


def _rne(a):
    b = lax.bitcast_convert_type(a, jnp.int32)
    r = (b + jnp.int32(0x7FFF) + ((b >> 16) & 1)) & jnp.int32(-65536)
    return lax.bitcast_convert_type(r, jnp.float32)


_P = 256


def _nbr_body(lin_smem, knots_smem, smx_smem, smy_smem,
              lin2_col, lin_ext, phiw, phii, phij,
              jout, prod_out, psis_out, phis_out):
    r0 = pl.program_id(0) * 4
    lin2 = lin2_col[...]
    oiota = lax.broadcasted_iota(jnp.int32, (_P, 128), 1)
    iiota = lax.broadcasted_iota(jnp.int32, (_P, 128), 0)
    odr = oiota // 11 - 5
    odc = oiota % 11 - 5
    ccol = iiota % _H
    qrow = iiota // _H
    lsh1 = jnp.concatenate([lin_ext[s:s + _H, :] for s in range(11)], axis=1)
    lin_sh = jnp.concatenate([lsh1, lsh1, lsh1, lsh1], axis=0)
    linj = jnp.zeros((_P, 128), jnp.float32)
    for s in range(11):
        linj = jnp.where(odc == s - 5, lin_sh[:, s:s + 1], linj)
    dy = lin2 - linj
    xiq = [lin_smem[r0 + q] for q in range(4)]
    q_c = qrow[:, 0:1]
    xi_col = jnp.where(q_c < 1, xiq[0],
                       jnp.where(q_c < 2, xiq[1],
                                 jnp.where(q_c < 3, xiq[2], xiq[3])))
    dxv = jnp.zeros((_P, 128), jnp.float32)
    for t in range(11):
        vals = [xiq[q] - lin_smem[jnp.clip(r0 + q + t - 5, 0, _H - 1)]
                for q in range(4)]
        vcol = jnp.where(q_c < 1, vals[0],
                         jnp.where(q_c < 2, vals[1],
                                   jnp.where(q_c < 3, vals[2], vals[3])))
        dxv = jnp.where(odr == t - 5, vcol, dxv)
    dist = jnp.sqrt(dxv * dxv + dy * dy)
    cc = ccol + odc
    rr = r0 + qrow + odr
    valid = ((cc >= 0) & (cc < _H) & (rr >= 0) & (rr < _H) & (oiota < 121))
    dist = jnp.where(valid, dist, jnp.inf)
    liota_f = oiota.astype(jnp.float32)
    sel = []
    for _ in range(_K):
        m = jnp.min(dist, axis=1, keepdims=True)
        cand = jnp.where(dist == m, liota_f, jnp.float32(1e9))
        amin = jnp.min(cand, axis=1, keepdims=True)
        sel.append(amin)
        dist = jnp.where(cand == amin, jnp.inf, dist)
    osel = jnp.concatenate(sel, axis=1).astype(jnp.int32)
    drsel = osel // 11 - 5
    dcsel = osel % 11 - 5
    ii_col = lax.broadcasted_iota(jnp.int32, (_P, 1), 0)
    q_col = ii_col // _H
    c_col = ii_col % _H
    jout[0] = (r0 + q_col + drsel) * _H + c_col + dcsel
    cjx = jnp.zeros((_P, _K), jnp.float32)
    cjy = jnp.zeros((_P, _K), jnp.float32)
    for t in range(11):
        vals = [lin_smem[jnp.clip(r0 + q + t - 5, 0, _H - 1)] for q in range(4)]
        vcol = jnp.where(q_c < 1, vals[0],
                         jnp.where(q_c < 2, vals[1],
                                   jnp.where(q_c < 3, vals[2], vals[3])))
        cjx = jnp.where(drsel == t - 5, vcol, cjx)
    for s in range(11):
        cjy = jnp.where(dcsel == s - 5, lin_sh[:, s:s + 1], cjy)
    relx = xi_col - cjx
    rely = lin2 - cjy
    relcat = jnp.concatenate([relx, rely], axis=1)
    kiota = lax.broadcasted_iota(jnp.int32, (1, 2 * _K), 1)
    psic = jnp.zeros((_P, 2 * _K), jnp.float32)
    for t in range(_NKNOTS):
        kv = knots_smem[t]
        wrow = jnp.where(kiota < _K, smx_smem[t], smy_smem[t])
        psic = psic + jnp.maximum(1.0 - jnp.abs(relcat - kv), 0.0) * wrow
    psi = psic[:, 0:_K] * psic[:, _K:2 * _K]
    rpw = _rne(phiw[...])
    rpi = _rne(phii[...])
    rpj = _rne(phij[...])
    i_e = _rne(xi_col) * rpi[0:1, :] + _rne(lin2) * rpi[1:2, :]
    j_e = (_rne(cjx)[:, :, None] * rpj[0:1, :].reshape(1, 1, _C)
           + _rne(cjy)[:, :, None] * rpj[1:2, :].reshape(1, 1, _C))
    t = i_e[:, None, :] * j_e
    phiv = jnp.sum(_rne(t) * rpw.reshape(1, 1, _C), axis=2)
    prod_out[0] = psi * phiv
    psis_out[0] = jnp.sum(jnp.abs(psi)).reshape(1, 1)
    phis_out[0] = jnp.sum(jnp.abs(phiv)).reshape(1, 1)


def _neighbors_and_weights(lin, knots, S_m_x, S_m_y, phi_w, phi_i, phi_j):
    nb = _H // 4
    out_shapes = [
        jax.ShapeDtypeStruct((nb, _P, _K), jnp.int32),
        jax.ShapeDtypeStruct((nb, _P, _K), jnp.float32),
        jax.ShapeDtypeStruct((nb, 1, 1), jnp.float32),
        jax.ShapeDtypeStruct((nb, 1, 1), jnp.float32),
    ]
    call = pl.pallas_call(
        _nbr_body,
        grid=(nb,),
        in_specs=[
            pl.BlockSpec(memory_space=pltpu.SMEM),
            pl.BlockSpec(memory_space=pltpu.SMEM),
            pl.BlockSpec(memory_space=pltpu.SMEM),
            pl.BlockSpec(memory_space=pltpu.SMEM),
            pl.BlockSpec((_P, 1), lambda r: (0, 0)),
            pl.BlockSpec((80, 1), lambda r: (0, 0)),
            pl.BlockSpec((1, _C), lambda r: (0, 0)),
            pl.BlockSpec((2, _C), lambda r: (0, 0)),
            pl.BlockSpec((2, _C), lambda r: (0, 0)),
        ],
        out_specs=[
            pl.BlockSpec((1, _P, _K), lambda r: (r, 0, 0)),
            pl.BlockSpec((1, _P, _K), lambda r: (r, 0, 0)),
            pl.BlockSpec((1, 1, 1), lambda r: (r, 0, 0)),
            pl.BlockSpec((1, 1, 1), lambda r: (r, 0, 0)),
        ],
        out_shape=out_shapes,
    )
    lin_ext = jnp.concatenate(
        [jnp.zeros((5,), jnp.float32), lin, jnp.zeros((11,), jnp.float32)])
    lin2 = jnp.concatenate([lin, lin, lin, lin]).reshape(_P, 1)
    return call(lin, knots, S_m_x, S_m_y, lin2,
                lin_ext.reshape(80, 1), phi_w.reshape(1, _C), phi_i.T,
                phi_j.T)


_G = 8
_NW = 32
_DPW = (2 * _N) // _NW
_NCH = _DPW // _G


def _agg_body(x_hbm, gidx_hbm, w_hbm, out_hbm,
              idx_v, w_v, rows0, rows1, out0, out1, sem0, sem1):
    cid = lax.axis_index("c")
    sid = lax.axis_index("s")
    wid = sid * 2 + cid
    pltpu.sync_copy(gidx_hbm.at[pl.ds(wid * (_DPW * _K // 128), _DPW * _K // 128), :],
                    idx_v)
    pltpu.sync_copy(w_hbm.at[pl.ds(wid * _DPW, _DPW), :], w_v)

    def issue(c, rows, sem):
        for j in range(2):
            pltpu.async_copy(x_hbm.at[idx_v.at[c * 2 + j]],
                             rows.at[pl.ds(j * 128, 128), :], sem)

    def drain(rows, sem):
        for j in range(2):
            pltpu.make_async_copy(x_hbm.at[pl.ds(0, 128), :],
                                  rows.at[pl.ds(j * 128, 128), :], sem).wait()

    def compute(c, rows, outv):
        def g_body(g, carry2):
            accs = [jnp.zeros((16,), jnp.float32) for _ in range(8)]
            wrow = c * _G + g
            whalf = (w_v[wrow, pl.ds(0, 16)], w_v[wrow, pl.ds(16, 16)])
            for k in range(_K):
                wb = lax.gather(
                    whalf[k // 16],
                    jnp.full((16, 1), k % 16, jnp.int32),
                    lax.GatherDimensionNumbers(
                        offset_dims=(), collapsed_slice_dims=(0,),
                        start_index_map=(0,)),
                    (1,),
                    mode=lax.GatherScatterMode.PROMISE_IN_BOUNDS)
                row = g * _K + k
                for c8 in range(8):
                    accs[c8] = accs[c8] + wb * rows[row, pl.ds(c8 * 16, 16)]
            for c8 in range(8):
                outv[g, pl.ds(c8 * 16, 16)] = accs[c8]
            return carry2

        lax.fori_loop(0, _G, g_body, 0)
        pltpu.sync_copy(outv, out_hbm.at[pl.ds(wid * _DPW + c * _G, _G), :])

    issue(0, rows0, sem0)

    def pair(i, carry):
        c0 = 2 * i
        issue(c0 + 1, rows1, sem1)
        drain(rows0, sem0)
        compute(c0, rows0, out0)

        @pl.when(i < _NCH // 2 - 1)
        def _():
            issue(c0 + 2, rows0, sem0)

        drain(rows1, sem1)
        compute(c0 + 1, rows1, out1)
        return carry

    lax.fori_loop(0, _NCH // 2, pair, 0)


def _aggregate(x_flat, gidx2d, w2):
    mesh = plsc.VectorSubcoreMesh(core_axis_name="c", subcore_axis_name="s")
    kern = functools.partial(
        pl.kernel,
        mesh=mesh,
        out_type=jax.ShapeDtypeStruct((2 * _N, _C), jnp.float32),
        scratch_types=[
            pltpu.VMEM((_DPW * _K // 128, 128), jnp.int32),
            pltpu.VMEM((_DPW, _K), jnp.float32),
            pltpu.VMEM((_G * _K, _C), jnp.float32),
            pltpu.VMEM((_G * _K, _C), jnp.float32),
            pltpu.VMEM((_G, _C), jnp.float32),
            pltpu.VMEM((_G, _C), jnp.float32),
            pltpu.SemaphoreType.DMA,
            pltpu.SemaphoreType.DMA,
        ],
    )(_agg_body)
    return kern(x_flat, gidx2d, w2)


def _mlp_body(scale_smem, x_ref, w1t_ref, b1_ref, w2t_ref, b2_ref, agg_ref,
              o_ref):
    h = jnp.dot(x_ref[...], w1t_ref[...], preferred_element_type=jnp.float32)
    h = jnp.maximum(h + b1_ref[...], 0.0)
    o = jnp.dot(h, w2t_ref[...], preferred_element_type=jnp.float32)
    o_ref[...] = o + b2_ref[...] + scale_smem[0] * agg_ref[...]


def _mlp_combine(x_flat, W1T, W1_b, W2T, W2_b, agg, scale):
    rows = 2 * _N
    br = 512
    return pl.pallas_call(
        _mlp_body,
        grid=(rows // br,),
        in_specs=[
            pl.BlockSpec(memory_space=pltpu.SMEM),
            pl.BlockSpec((br, _C), lambda r: (r, 0)),
            pl.BlockSpec((_C, 2 * _C), lambda r: (0, 0)),
            pl.BlockSpec((1, 2 * _C), lambda r: (0, 0)),
            pl.BlockSpec((2 * _C, _C), lambda r: (0, 0)),
            pl.BlockSpec((1, _C), lambda r: (0, 0)),
            pl.BlockSpec((br, _C), lambda r: (r, 0)),
        ],
        out_specs=pl.BlockSpec((br, _C), lambda r: (r, 0)),
        out_shape=jax.ShapeDtypeStruct((rows, _C), jnp.float32),
    )(scale, x_flat, W1T, W1_b.reshape(1, 2 * _C), W2T, W2_b.reshape(1, _C),
      agg)


def kernel(x, W1_w, W1_b, W2_w, W2_b, phi_w, phi_i, phi_j,
           h1_w, h1_b, h2_w, h2_b, S_m_x, S_m_y):
    lin = jnp.linspace(0.0, 1.0, _H).astype(jnp.float32)
    knots = jnp.linspace(0.0, 1.0, _NKNOTS).astype(jnp.float32)
    jout, prod, psis, phis = _neighbors_and_weights(
        lin, knots, S_m_x, S_m_y, phi_w, phi_i, phi_j)
    jflat = jout.reshape(_N, _K)
    prod_flat = prod.reshape(_N, _K)
    mpsi = jnp.sum(psis) / (_N * _K)
    mphi = jnp.sum(phis) / (_N * _K)
    scale = 1.0 / ((mpsi + 1e-6) * (mphi + 1e-6) * jnp.float32(_K))
    gidx = jnp.concatenate([jflat, jflat + _N], axis=0)
    gidx = gidx.reshape((2 * _N * _K) // 128, 128)
    w2 = jnp.concatenate([prod_flat, prod_flat], axis=0)
    x_flat = x.reshape(2 * _N, _C)
    agg = _aggregate(x_flat, gidx, w2)
    out = _mlp_combine(x_flat, W1_w.T, W1_b, W2_w.T, W2_b, agg,
                       scale.reshape(1))
    return out.reshape(2, _N, _C)
